# Initial kernel scaffold; baseline (speedup 1.0000x reference)
import jax
import jax.numpy as jnp
from jax.experimental import pallas as pl


def kernel(inputs, W0, b0, W1, b1, W2, b2):
    raise NotImplementedError("write your pallas kernel here")


# speedup vs baseline: 7.2405x; 7.2405x over previous
import functools

import jax
import jax.numpy as jnp
from jax.experimental import pallas as pl
from jax.experimental.pallas import tpu as pltpu

C = 128
G = 64
K = 16


def _edgeconv_kernel(x_ref, wcat_ref, w1a_ref, w2a_ref, w2b_ref, bias_ref,
                     out_ref, t_ref, sqa_ref, d_ref, *, bn, n):
    nb = pl.program_id(1)
    xa = x_ref[0]

    @pl.when(nb == 0)
    def _():
        t_ref[...] = jnp.dot(xa, wcat_ref[...],
                             preferred_element_type=jnp.float32)
        sqa_ref[...] = jnp.sum(xa * xa, axis=1, keepdims=True).reshape(1, n)

    xb = x_ref[0, pl.ds(nb * bn, bn), :]
    tb = t_ref[pl.ds(nb * bn, bn), :]

    sqb = jnp.sum(xb * xb, axis=1, keepdims=True)
    cross = jax.lax.dot_general(xb, xa, (((1,), (1,)), ((), ())),
                                preferred_element_type=jnp.float32)
    d = sqb - 2.0 * cross + sqa_ref[...]

    cols = jax.lax.broadcasted_iota(jnp.int32, (bn, n), 1)
    rows_g = nb * bn + jax.lax.broadcasted_iota(jnp.int32, (bn, n), 0)
    inf = jnp.float32(jnp.inf)
    d_ref[...] = jnp.where(cols == rows_g, inf, d)

    p_all = t_ref[:, 0:G]
    pb = tb[:, 0:G]
    qb = tb[:, G:2 * G]
    c1b = tb[:, 2 * G:3 * G]
    c2b = tb[:, 3 * G:4 * G]
    b0 = bias_ref[0, :][None, :]
    b1 = bias_ref[1, :][None, :]
    b2 = bias_ref[2, :][None, :]
    base0 = qb - pb + b0
    c1bb = c1b + b1

    w1a = w1a_ref[...]
    w2a = w2a_ref[...]
    w2b = w2b_ref[...]

    neg = jnp.float32(-jnp.inf)
    m0 = jnp.full((bn, G), neg, jnp.float32)
    m1 = jnp.full((bn, G), neg, jnp.float32)
    m2 = jnp.full((bn, G), neg, jnp.float32)

    def body(_, carry):
        m0, m1, m2 = carry
        dv = d_ref[...]
        m = jnp.min(dv, axis=1, keepdims=True)
        idx = jnp.min(jnp.where(dv == m, cols, n), axis=1, keepdims=True)
        onehot = cols == idx
        d_ref[...] = jnp.where(onehot, inf, dv)
        pg = jnp.dot(onehot.astype(jnp.float32), p_all,
                     preferred_element_type=jnp.float32)
        h0 = jnp.maximum(base0 + pg, 0.0)
        h1 = jnp.maximum(
            jnp.dot(h0, w1a, preferred_element_type=jnp.float32) + c1bb, 0.0)
        t2 = (jnp.dot(h1, w2a, preferred_element_type=jnp.float32)
              + jnp.dot(h0, w2b, preferred_element_type=jnp.float32))
        return (jnp.maximum(m0, h0), jnp.maximum(m1, h1), jnp.maximum(m2, t2))

    m0, m1, m2 = jax.lax.fori_loop(0, K, body, (m0, m1, m2))

    out_ref[0, :, 0:G] = m2 + c2b + b2
    out_ref[0, :, G:2 * G] = m1
    out_ref[0, :, 2 * G:3 * G] = m0
    out_ref[0, :, 3 * G:3 * G + C] = xb


def kernel(inputs, W0, b0, W1, b1, W2, b2):
    x = inputs
    B, N, _ = x.shape
    BN = 256 if N % 256 == 0 else N
    nblk = N // BN

    wcat = jnp.concatenate([W0[C:2 * C], W0[0:C], W1[G:G + C], W2[2 * G:]],
                           axis=1)
    bias = jnp.zeros((8, G), jnp.float32)
    bias = bias.at[0].set(b0).at[1].set(b1).at[2].set(b2)

    grid = (B, nblk)
    out = pl.pallas_call(
        functools.partial(_edgeconv_kernel, bn=BN, n=N),
        grid=grid,
        in_specs=[
            pl.BlockSpec((1, N, C), lambda b, nb: (b, 0, 0)),
            pl.BlockSpec((C, 4 * G), lambda b, nb: (0, 0)),
            pl.BlockSpec((G, G), lambda b, nb: (0, 0)),
            pl.BlockSpec((G, G), lambda b, nb: (0, 0)),
            pl.BlockSpec((G, G), lambda b, nb: (0, 0)),
            pl.BlockSpec((8, G), lambda b, nb: (0, 0)),
        ],
        out_specs=pl.BlockSpec((1, BN, 3 * G + C), lambda b, nb: (b, nb, 0)),
        out_shape=jax.ShapeDtypeStruct((B, N, 3 * G + C), jnp.float32),
        scratch_shapes=[
            pltpu.VMEM((N, 4 * G), jnp.float32),
            pltpu.VMEM((1, N), jnp.float32),
            pltpu.VMEM((BN, N), jnp.float32),
        ],
    )(x, wcat, W1[:G], W2[:G], W2[G:2 * G], bias)
    return out


# speedup vs baseline: 15.6035x; 2.1550x over previous
import functools

import jax
import jax.numpy as jnp
from jax.experimental import pallas as pl
from jax.experimental.pallas import tpu as pltpu

C = 128
G = 64
K = 16


def _nt(a, b):
    return jax.lax.dot_general(a, b, (((1,), (1,)), ((), ())),
                               preferred_element_type=jnp.float32)


def _nn(a, b):
    return jax.lax.dot_general(a, b, (((1,), (0,)), ((), ())),
                               preferred_element_type=jnp.float32)


def _edgeconv_kernel(x_ref, wcatt_ref, w1at_ref, w2at_ref, w2bt_ref,
                     biast_ref, ones_ref, out_ref, tt_ref, sqa_ref, dt_ref,
                     *, bn, n):
    nb = pl.program_id(1)
    xa = x_ref[0]

    @pl.when(nb == 0)
    def _():
        tt_ref[...] = _nt(wcatt_ref[...], xa)
        sqa_ref[...] = jnp.sum(xa * xa, axis=1, keepdims=True)

    xb = x_ref[0, pl.ds(nb * bn, bn), :]
    sqb_row = _nt(ones_ref[...], xb * xb)

    dt = sqa_ref[...] - 2.0 * _nt(xa, xb) + sqb_row

    rows_n = jax.lax.broadcasted_iota(jnp.int32, (n, bn), 0)
    cols_b = jax.lax.broadcasted_iota(jnp.int32, (n, bn), 1)
    inf = jnp.float32(jnp.inf)
    dt_ref[...] = jnp.where(rows_n == nb * bn + cols_b, inf, dt)

    cols = pl.ds(nb * bn, bn)
    p_all_t = tt_ref[0:G, :]
    pbt = tt_ref[0:G, cols]
    qbt = tt_ref[G:2 * G, cols]
    c1bt = tt_ref[2 * G:3 * G, cols]
    c2bt = tt_ref[3 * G:4 * G, cols]
    b0 = biast_ref[:, 0:1]
    b1 = biast_ref[:, 1:2]
    b2 = biast_ref[:, 2:3]
    base0 = qbt - pbt + b0
    c1bb = c1bt + b1

    w1at = w1at_ref[...]
    w2at = w2at_ref[...]
    w2bt = w2bt_ref[...]

    neg = jnp.float32(-jnp.inf)
    m0 = jnp.full((G, bn), neg, jnp.float32)
    m1 = jnp.full((G, bn), neg, jnp.float32)
    m2 = jnp.full((G, bn), neg, jnp.float32)

    for _ in range(K):
        dv = dt_ref[...]
        idx = jnp.argmin(dv, axis=0)[None, :]
        onehot = rows_n == idx
        dt_ref[...] = jnp.where(onehot, inf, dv)
        pg = _nn(p_all_t, jnp.where(onehot, 1.0, 0.0))
        h0 = jnp.maximum(base0 + pg, 0.0)
        h1 = jnp.maximum(_nn(w1at, h0) + c1bb, 0.0)
        t2 = _nn(w2at, h1) + _nn(w2bt, h0)
        m0 = jnp.maximum(m0, h0)
        m1 = jnp.maximum(m1, h1)
        m2 = jnp.maximum(m2, t2)

    hcat = jnp.concatenate([m2 + c2bt + b2, m1, m0], axis=0)
    out_ref[0, :, 0:3 * G] = jnp.transpose(hcat)
    out_ref[0, :, 3 * G:3 * G + C] = xb


def kernel(inputs, W0, b0, W1, b1, W2, b2):
    x = inputs
    B, N, _ = x.shape
    BN = 256 if N % 256 == 0 else N
    nblk = N // BN

    wcat_t = jnp.concatenate([W0[C:2 * C], W0[0:C], W1[G:G + C], W2[2 * G:]],
                             axis=1).T
    bias_t = jnp.zeros((G, 8), jnp.float32)
    bias_t = bias_t.at[:, 0].set(b0).at[:, 1].set(b1).at[:, 2].set(b2)
    ones_c = jnp.ones((1, C), jnp.float32)

    grid = (B, nblk)
    out = pl.pallas_call(
        functools.partial(_edgeconv_kernel, bn=BN, n=N),
        grid=grid,
        in_specs=[
            pl.BlockSpec((1, N, C), lambda b, nb: (b, 0, 0)),
            pl.BlockSpec((4 * G, C), lambda b, nb: (0, 0)),
            pl.BlockSpec((G, G), lambda b, nb: (0, 0)),
            pl.BlockSpec((G, G), lambda b, nb: (0, 0)),
            pl.BlockSpec((G, G), lambda b, nb: (0, 0)),
            pl.BlockSpec((G, 8), lambda b, nb: (0, 0)),
            pl.BlockSpec((1, C), lambda b, nb: (0, 0)),
        ],
        out_specs=pl.BlockSpec((1, BN, 3 * G + C), lambda b, nb: (b, nb, 0)),
        out_shape=jax.ShapeDtypeStruct((B, N, 3 * G + C), jnp.float32),
        scratch_shapes=[
            pltpu.VMEM((4 * G, N), jnp.float32),
            pltpu.VMEM((N, 1), jnp.float32),
            pltpu.VMEM((N, BN), jnp.float32),
        ],
    )(x, wcat_t, W1[:G].T, W2[:G].T, W2[G:2 * G].T, bias_t, ones_c)
    return out
